# baseline (speedup 1.0000x reference)
import dataclasses
import functools

import jax
import jax.numpy as jnp
from jax import lax
from jax.experimental import pallas as pl
from jax.experimental.pallas import tpu as pltpu
from jax.experimental.pallas import tpu_sc as plsc

NC = 2
NS = 16
NW = NC * NS
L16 = 16

HID = 64
KV = HID // L16

DEMO_VOCAB = 128
MAX_POS = 512

C = 128
NS5 = 5
NB = 6


def _rsqrt(x):
    i = lax.bitcast_convert_type(x, jnp.int32)
    i = jnp.int32(0x5F375A86) - lax.shift_right_arithmetic(i, 1)
    y = lax.bitcast_convert_type(i, jnp.float32)
    return y * (1.5 - 0.5 * x * y * y)


def _xorp(v, iota, kbit):
    return v.at[jnp.bitwise_xor(iota, jnp.int32(kbit))].get(
        mode="promise_in_bounds")


def _bcast(vec, j):
    return vec.at[jnp.full((L16,), j, jnp.int32)].get(
        mode="promise_in_bounds")


@functools.partial(jax.jit, static_argnames=("n_tok",))
def _embed_ln(n_tok, idw2, ids5, wtab, dtab_f, ptab_f, stab_f, gamma, beta):
    tok_w = n_tok // NW
    nchunk = tok_w // C
    assert nchunk % 2 == 0 and nchunk >= NB + 2
    rows_w = nchunk
    n_rows = n_tok // C
    mesh = plsc.VectorSubcoreMesh(core_axis_name="c", subcore_axis_name="s")
    cp = pltpu.CompilerParams()
    if "needs_layout_passes" in pltpu.CompilerParams.__dataclass_fields__:
        cp = dataclasses.replace(cp, needs_layout_passes=False)
    if "use_tc_tiling_on_sc" in pltpu.CompilerParams.__dataclass_fields__:
        cp = dataclasses.replace(cp, use_tc_tiling_on_sc=False)

    @functools.partial(
        pl.kernel,
        compiler_params=cp,
        out_type=jax.ShapeDtypeStruct((n_rows, C, HID), jnp.float32),
        mesh=mesh,
        scratch_types=[
            pltpu.VMEM((4 * rows_w, C // 4), jnp.int32),
            pltpu.VMEM((NS5 * rows_w, C), jnp.int32),
            pltpu.VMEM((C, HID), jnp.float32),
            pltpu.VMEM((C, HID), jnp.float32),
            pltpu.VMEM((C, HID), jnp.float32),
            pltpu.VMEM((C, HID), jnp.float32),
            pltpu.VMEM((C, HID), jnp.float32),
            pltpu.VMEM((C, HID), jnp.float32),
            pltpu.VMEM((DEMO_VOCAB * HID,), jnp.float32),
            pltpu.VMEM((MAX_POS * HID,), jnp.float32),
            pltpu.VMEM((2 * HID,), jnp.float32),
            pltpu.VMEM((HID,), jnp.float32),
            pltpu.VMEM((HID,), jnp.float32),
            pltpu.SemaphoreType.DMA,
            pltpu.SemaphoreType.DMA,
            pltpu.SemaphoreType.DMA,
            pltpu.SemaphoreType.DMA,
            pltpu.SemaphoreType.DMA,
            pltpu.SemaphoreType.DMA,
            pltpu.SemaphoreType.DMA,
            pltpu.SemaphoreType.DMA,
            pltpu.SemaphoreType.DMA,
            pltpu.SemaphoreType.DMA,
            pltpu.SemaphoreType.DMA,
            pltpu.SemaphoreType.DMA,
        ],
    )
    def k(idw2_h, ids5_h, wtab_h, dtab_h, ptab_h, stab_h, gamma_h, beta_h,
          out_h,
          idwb, idsb, wr0, wr1, wr2, wr3, wr4, wr5,
          dtab_v, ptab_v, stab_v, g_v, b_v,
          sg0, sg1, sg2, sg3, sg4, sg5, so0, so1, so2, so3, so4, so5):
        wid = lax.axis_index("s") * NC + lax.axis_index("c")
        row0 = wid * rows_w

        pltpu.sync_copy(dtab_h, dtab_v)
        pltpu.sync_copy(ptab_h, ptab_v)
        pltpu.sync_copy(stab_h, stab_v)
        pltpu.sync_copy(gamma_h, g_v)
        pltpu.sync_copy(beta_h, b_v)
        pltpu.sync_copy(idw2_h.at[wid], idwb)
        pltpu.sync_copy(ids5_h.at[wid], idsb)

        wrs = (wr0, wr1, wr2, wr3, wr4, wr5)
        sem_g = (sg0, sg1, sg2, sg3, sg4, sg5)
        sem_o = (so0, so1, so2, so3, so4, so5)
        AGE, BMI, CYC, SEG, POS = range(NS5)
        H4 = C // 4

        def issue_word(g, p):
            for q in range(4):
                pltpu.async_copy(
                    wtab_h.at[idwb.at[4 * g + q]],
                    wrs[p].at[pl.ds(q * H4, H4)], sem_g[p])

        def wait_word(g, p):
            for q in range(4):
                pltpu.make_async_copy(
                    wtab_h.at[idwb.at[4 * g + q]],
                    wrs[p].at[pl.ds(q * H4, H4)], sem_g[p]).wait()

        def issue_flush(g, p):
            pltpu.async_copy(wrs[p], out_h.at[row0 + g], sem_o[p])

        def wait_flush(p):
            pltpu.make_async_copy(wrs[p], out_h.at[row0], sem_o[p]).wait()

        def compute(g, p):
            wr = wrs[p]
            iota = lax.iota(jnp.int32, L16)
            cvec = [kk * L16 + iota for kk in range(KV)]
            lo8 = iota < 8
            gvec = [g_v[pl.ds(kk * L16, L16)] for kk in range(KV)]
            bvec = [b_v[pl.ds(kk * L16, L16)] for kk in range(KV)]
            s0vec = [stab_v[pl.ds(kk * L16, L16)] for kk in range(KV)]
            sdvec = [stab_v[pl.ds(HID + kk * L16, L16)] - s0vec[kk]
                     for kk in range(KV)]

            def grow(tab_v, idv, j):
                base = _bcast(idv, j)
                return [plsc.load_gather(tab_v, [base + cvec[kk]])
                        for kk in range(KV)]

            def embed(av, bv, cv, pv, sv, t, j):
                ar = grow(dtab_v, av, j)
                br = grow(dtab_v, bv, j)
                cr = grow(dtab_v, cv, j)
                pr = grow(ptab_v, pv, j)
                segf = _bcast(sv, j).astype(jnp.float32)
                acc = []
                for kk in range(KV):
                    v = ((wr[t, pl.ds(kk * L16, L16)] + ar[kk])
                         + (br[kk] + cr[kk])
                         + (pr[kk] + (s0vec[kk] + segf * sdvec[kk])))
                    acc.append(v)
                s1 = (acc[0] + acc[1]) + (acc[2] + acc[3])
                sq = ((acc[0] * acc[0] + acc[1] * acc[1])
                      + (acc[2] * acc[2] + acc[3] * acc[3]))
                return acc, s1, sq

            def fold2(xa, xb):
                m = jnp.where(lo8, xa + _xorp(xa, iota, 8),
                              xb + _xorp(xb, iota, 8))
                for kbit in (4, 2, 1):
                    m = m + _xorp(m, iota, kbit)
                return m

            @pl.loop(0, C // L16)
            def _grp(gg):
                s = gg * L16
                av = idsb[AGE * rows_w + g, pl.ds(s, L16)]
                bv = idsb[BMI * rows_w + g, pl.ds(s, L16)]
                cv = idsb[CYC * rows_w + g, pl.ds(s, L16)]
                sv = idsb[SEG * rows_w + g, pl.ds(s, L16)]
                pv = idsb[POS * rows_w + g, pl.ds(s, L16)]

                for j2 in range(L16 // 2):
                    ta, tb = s + 2 * j2, s + 2 * j2 + 1
                    acc_a, s1a, sqa = embed(av, bv, cv, pv, sv, ta, 2 * j2)
                    acc_b, s1b, sqb = embed(av, bv, cv, pv, sv, tb,
                                            2 * j2 + 1)
                    su = fold2(s1a, s1b)
                    qu = fold2(sqa, sqb)
                    mn = su * (1.0 / HID)
                    var = qu * (1.0 / HID) - mn * mn
                    rs = _rsqrt(var + 1e-12)
                    m_a, m_b = _bcast(mn, 0), _bcast(mn, 8)
                    r_a, r_b = _bcast(rs, 0), _bcast(rs, 8)
                    for kk in range(KV):
                        wr[ta, pl.ds(kk * L16, L16)] = (
                            (acc_a[kk] - m_a) * (r_a * gvec[kk]) + bvec[kk])
                        wr[tb, pl.ds(kk * L16, L16)] = (
                            (acc_b[kk] - m_b) * (r_b * gvec[kk]) + bvec[kk])

            del _grp

        def do_chunk(g, p, pr, steady):
            wait_word(g, p)
            compute(g, p)
            issue_flush(g, p)
            if steady:
                @pl.when(jnp.logical_and(g >= 1, g + NB - 1 < nchunk))
                def _():
                    wait_flush(pr)

                @pl.when(g + NB - 1 < nchunk)
                def _():
                    issue_word(g + NB - 1, pr)

        for p in range(NB - 1):
            issue_word(p, p)

        @pl.loop(0, (nchunk - 2) // NB)
        def _ring(i):
            g = i * NB
            for p in range(NB):
                do_chunk(g + p, p, (p - 1) % NB, True)

        do_chunk(nchunk - 2, 0, NB - 1, False)
        do_chunk(nchunk - 1, 1, NB - 1, False)

        for p in range(2, NB):
            wait_flush(p)
        wait_flush(0)
        wait_flush(1)

    return k(idw2, ids5, wtab, dtab_f, ptab_f, stab_f, gamma, beta)


def kernel(word_ids, age_ids, bmi_ids, cycle_len_ids, seg_ids, posi_ids,
           word_table, demo_table, posi_table, seg_table, ln_gamma, ln_beta):
    b, l = word_ids.shape
    n_tok = b * l
    rows_w = n_tok // (NW * C)
    idw2 = word_ids.reshape(NW, 4 * rows_w, C // 4).astype(jnp.int32)
    as_w = lambda x: x.reshape(NW, rows_w, C).astype(jnp.int32)
    ids5 = jnp.stack(
        [as_w(age_ids) * HID, as_w(bmi_ids) * HID,
         as_w(cycle_len_ids) * HID, as_w(seg_ids),
         as_w(posi_ids) * HID],
        axis=1).reshape(NW, NS5 * rows_w, C)
    out = _embed_ln(
        n_tok, idw2, ids5,
        word_table.astype(jnp.float32),
        demo_table.astype(jnp.float32).reshape(-1),
        posi_table.astype(jnp.float32).reshape(-1),
        seg_table.astype(jnp.float32).reshape(-1),
        ln_gamma.astype(jnp.float32), ln_beta.astype(jnp.float32),
    )
    return out.reshape(b, l, HID)


# speedup vs baseline: 1.0070x; 1.0070x over previous
import dataclasses
import functools

import jax
import jax.numpy as jnp
from jax import lax
from jax.experimental import pallas as pl
from jax.experimental.pallas import tpu as pltpu
from jax.experimental.pallas import tpu_sc as plsc

NC = 2
NS = 16
NW = NC * NS
L16 = 16

HID = 64
KV = HID // L16

DEMO_VOCAB = 128
MAX_POS = 512

C = 128
NS5 = 5
NB = 4


def _rsqrt(x):
    i = lax.bitcast_convert_type(x, jnp.int32)
    i = jnp.int32(0x5F375A86) - lax.shift_right_arithmetic(i, 1)
    y = lax.bitcast_convert_type(i, jnp.float32)
    return y * (1.5 - 0.5 * x * y * y)


def _xorp(v, iota, kbit):
    return v.at[jnp.bitwise_xor(iota, jnp.int32(kbit))].get(
        mode="promise_in_bounds")


def _bcast(vec, j):
    return vec.at[jnp.full((L16,), j, jnp.int32)].get(
        mode="promise_in_bounds")


@functools.partial(jax.jit, static_argnames=("n_tok",))
def _embed_ln(n_tok, idw2, ids5, wtab, dtab_f, ptab_f, stab_f, gamma, beta):
    tok_w = n_tok // NW
    nchunk = tok_w // C
    assert nchunk % 2 == 0 and nchunk >= NB + 2
    rows_w = nchunk
    n_rows = n_tok // C
    mesh = plsc.VectorSubcoreMesh(core_axis_name="c", subcore_axis_name="s")
    cp = pltpu.CompilerParams()
    if "needs_layout_passes" in pltpu.CompilerParams.__dataclass_fields__:
        cp = dataclasses.replace(cp, needs_layout_passes=False)
    if "use_tc_tiling_on_sc" in pltpu.CompilerParams.__dataclass_fields__:
        cp = dataclasses.replace(cp, use_tc_tiling_on_sc=False)

    @functools.partial(
        pl.kernel,
        compiler_params=cp,
        out_type=jax.ShapeDtypeStruct((n_rows, C, HID), jnp.float32),
        mesh=mesh,
        scratch_types=[
            pltpu.VMEM((8 * rows_w, C // 8), jnp.int32),
            pltpu.VMEM((NS5 * rows_w, C), jnp.int32),
            pltpu.VMEM((C, HID), jnp.float32),
            pltpu.VMEM((C, HID), jnp.float32),
            pltpu.VMEM((C, HID), jnp.float32),
            pltpu.VMEM((C, HID), jnp.float32),
            pltpu.VMEM((DEMO_VOCAB * HID,), jnp.float32),
            pltpu.VMEM((MAX_POS * HID,), jnp.float32),
            pltpu.VMEM((2 * HID,), jnp.float32),
            pltpu.VMEM((HID,), jnp.float32),
            pltpu.VMEM((HID,), jnp.float32),
            pltpu.SemaphoreType.DMA,
            pltpu.SemaphoreType.DMA,
            pltpu.SemaphoreType.DMA,
            pltpu.SemaphoreType.DMA,
            pltpu.SemaphoreType.DMA,
            pltpu.SemaphoreType.DMA,
            pltpu.SemaphoreType.DMA,
            pltpu.SemaphoreType.DMA,
        ],
    )
    def k(idw2_h, ids5_h, wtab_h, dtab_h, ptab_h, stab_h, gamma_h, beta_h,
          out_h,
          idwb, idsb, wr0, wr1, wr2, wr3, dtab_v, ptab_v, stab_v, g_v, b_v,
          sg0, sg1, sg2, sg3, so0, so1, so2, so3):
        wid = lax.axis_index("s") * NC + lax.axis_index("c")
        row0 = wid * rows_w

        pltpu.sync_copy(dtab_h, dtab_v)
        pltpu.sync_copy(ptab_h, ptab_v)
        pltpu.sync_copy(stab_h, stab_v)
        pltpu.sync_copy(gamma_h, g_v)
        pltpu.sync_copy(beta_h, b_v)
        pltpu.sync_copy(idw2_h.at[wid], idwb)
        pltpu.sync_copy(ids5_h.at[wid], idsb)

        wrs = (wr0, wr1, wr2, wr3)
        sem_g = (sg0, sg1, sg2, sg3)
        sem_o = (so0, so1, so2, so3)
        AGE, BMI, CYC, SEG, POS = range(NS5)
        H8 = C // 8

        def issue_word(g, p):
            for q in range(8):
                pltpu.async_copy(
                    wtab_h.at[idwb.at[8 * g + q]],
                    wrs[p].at[pl.ds(q * H8, H8)], sem_g[p])

        def wait_word(g, p):
            for q in range(8):
                pltpu.make_async_copy(
                    wtab_h.at[idwb.at[8 * g + q]],
                    wrs[p].at[pl.ds(q * H8, H8)], sem_g[p]).wait()

        def issue_flush(g, p):
            pltpu.async_copy(wrs[p], out_h.at[row0 + g], sem_o[p])

        def wait_flush(p):
            pltpu.make_async_copy(wrs[p], out_h.at[row0], sem_o[p]).wait()

        def compute(g, p):
            wr = wrs[p]
            iota = lax.iota(jnp.int32, L16)
            cvec = [kk * L16 + iota for kk in range(KV)]
            lo8 = iota < 8
            gvec = [g_v[pl.ds(kk * L16, L16)] for kk in range(KV)]
            bvec = [b_v[pl.ds(kk * L16, L16)] for kk in range(KV)]
            s0vec = [stab_v[pl.ds(kk * L16, L16)] for kk in range(KV)]
            sdvec = [stab_v[pl.ds(HID + kk * L16, L16)] - s0vec[kk]
                     for kk in range(KV)]

            def grow(tab_v, idv, j):
                base = _bcast(idv, j)
                return [plsc.load_gather(tab_v, [base + cvec[kk]])
                        for kk in range(KV)]

            def embed(av, bv, cv, pv, sv, t, j):
                ar = grow(dtab_v, av, j)
                br = grow(dtab_v, bv, j)
                cr = grow(dtab_v, cv, j)
                pr = grow(ptab_v, pv, j)
                segf = _bcast(sv, j).astype(jnp.float32)
                acc = []
                for kk in range(KV):
                    v = ((wr[t, pl.ds(kk * L16, L16)] + ar[kk])
                         + (br[kk] + cr[kk])
                         + (pr[kk] + (s0vec[kk] + segf * sdvec[kk])))
                    acc.append(v)
                s1 = (acc[0] + acc[1]) + (acc[2] + acc[3])
                sq = ((acc[0] * acc[0] + acc[1] * acc[1])
                      + (acc[2] * acc[2] + acc[3] * acc[3]))
                return acc, s1, sq

            def fold2(xa, xb):
                m = jnp.where(lo8, xa + _xorp(xa, iota, 8),
                              xb + _xorp(xb, iota, 8))
                for kbit in (4, 2, 1):
                    m = m + _xorp(m, iota, kbit)
                return m

            @pl.loop(0, C // L16)
            def _grp(gg):
                s = gg * L16
                av = idsb[AGE * rows_w + g, pl.ds(s, L16)]
                bv = idsb[BMI * rows_w + g, pl.ds(s, L16)]
                cv = idsb[CYC * rows_w + g, pl.ds(s, L16)]
                sv = idsb[SEG * rows_w + g, pl.ds(s, L16)]
                pv = idsb[POS * rows_w + g, pl.ds(s, L16)]

                for j2 in range(L16 // 2):
                    ta, tb = s + 2 * j2, s + 2 * j2 + 1
                    acc_a, s1a, sqa = embed(av, bv, cv, pv, sv, ta, 2 * j2)
                    acc_b, s1b, sqb = embed(av, bv, cv, pv, sv, tb,
                                            2 * j2 + 1)
                    su = fold2(s1a, s1b)
                    qu = fold2(sqa, sqb)
                    mn = su * (1.0 / HID)
                    var = qu * (1.0 / HID) - mn * mn
                    rs = _rsqrt(var + 1e-12)
                    m_a, m_b = _bcast(mn, 0), _bcast(mn, 8)
                    r_a, r_b = _bcast(rs, 0), _bcast(rs, 8)
                    for kk in range(KV):
                        wr[ta, pl.ds(kk * L16, L16)] = (
                            (acc_a[kk] - m_a) * (r_a * gvec[kk]) + bvec[kk])
                        wr[tb, pl.ds(kk * L16, L16)] = (
                            (acc_b[kk] - m_b) * (r_b * gvec[kk]) + bvec[kk])

            del _grp

        def do_chunk(g, p, pr, steady):
            wait_word(g, p)
            compute(g, p)
            issue_flush(g, p)
            if steady:
                @pl.when(jnp.logical_and(g >= 1, g + NB - 1 < nchunk))
                def _():
                    wait_flush(pr)

                @pl.when(g + NB - 1 < nchunk)
                def _():
                    issue_word(g + NB - 1, pr)

        for p in range(NB - 1):
            issue_word(p, p)

        @pl.loop(0, (nchunk - 2) // NB)
        def _ring(i):
            g = i * NB
            for p in range(NB):
                do_chunk(g + p, p, (p - 1) % NB, True)

        do_chunk(nchunk - 2, 0, NB - 1, False)
        do_chunk(nchunk - 1, 1, NB - 1, False)

        for p in range(2, NB):
            wait_flush(p)
        wait_flush(0)
        wait_flush(1)

    return k(idw2, ids5, wtab, dtab_f, ptab_f, stab_f, gamma, beta)


def kernel(word_ids, age_ids, bmi_ids, cycle_len_ids, seg_ids, posi_ids,
           word_table, demo_table, posi_table, seg_table, ln_gamma, ln_beta):
    b, l = word_ids.shape
    n_tok = b * l
    rows_w = n_tok // (NW * C)
    idw2 = word_ids.reshape(NW, 8 * rows_w, C // 8).astype(jnp.int32)
    as_w = lambda x: x.reshape(NW, rows_w, C).astype(jnp.int32)
    ids5 = jnp.stack(
        [as_w(age_ids) * HID, as_w(bmi_ids) * HID,
         as_w(cycle_len_ids) * HID, as_w(seg_ids),
         as_w(posi_ids) * HID],
        axis=1).reshape(NW, NS5 * rows_w, C)
    out = _embed_ln(
        n_tok, idw2, ids5,
        word_table.astype(jnp.float32),
        demo_table.astype(jnp.float32).reshape(-1),
        posi_table.astype(jnp.float32).reshape(-1),
        seg_table.astype(jnp.float32).reshape(-1),
        ln_gamma.astype(jnp.float32), ln_beta.astype(jnp.float32),
    )
    return out.reshape(b, l, HID)


# speedup vs baseline: 1.0088x; 1.0018x over previous
import dataclasses
import functools

import jax
import jax.numpy as jnp
from jax import lax
from jax.experimental import pallas as pl
from jax.experimental.pallas import tpu as pltpu
from jax.experimental.pallas import tpu_sc as plsc

NC = 2
NS = 16
NW = NC * NS
L16 = 16

HID = 64
KV = HID // L16

DEMO_VOCAB = 128
MAX_POS = 512

C = 128
NS5 = 5
NB = 4


def _rsqrt(x):
    i = lax.bitcast_convert_type(x, jnp.int32)
    i = jnp.int32(0x5F375A86) - lax.shift_right_arithmetic(i, 1)
    y = lax.bitcast_convert_type(i, jnp.float32)
    return y * (1.5 - 0.5 * x * y * y)


def _xorp(v, iota, kbit):
    return v.at[jnp.bitwise_xor(iota, jnp.int32(kbit))].get(
        mode="promise_in_bounds")


def _bcast(vec, j):
    return vec.at[jnp.full((L16,), j, jnp.int32)].get(
        mode="promise_in_bounds")


@functools.partial(jax.jit, static_argnames=("n_tok",))
def _embed_ln(n_tok, idw2, ids5, wtab, dtab_f, ptab_f, stab_f, gamma, beta):
    tok_w = n_tok // NW
    nchunk = tok_w // C
    assert nchunk % 2 == 0 and nchunk >= NB + 2
    rows_w = nchunk
    n_rows = n_tok // C
    mesh = plsc.VectorSubcoreMesh(core_axis_name="c", subcore_axis_name="s")
    cp = pltpu.CompilerParams()
    if "needs_layout_passes" in pltpu.CompilerParams.__dataclass_fields__:
        cp = dataclasses.replace(cp, needs_layout_passes=False)
    if "use_tc_tiling_on_sc" in pltpu.CompilerParams.__dataclass_fields__:
        cp = dataclasses.replace(cp, use_tc_tiling_on_sc=False)

    @functools.partial(
        pl.kernel,
        compiler_params=cp,
        out_type=jax.ShapeDtypeStruct((n_rows, C, HID), jnp.float32),
        mesh=mesh,
        scratch_types=[
            pltpu.VMEM((4 * rows_w, C // 4), jnp.int32),
            pltpu.VMEM((NS5 * rows_w, C), jnp.int32),
            pltpu.VMEM((C, HID), jnp.float32),
            pltpu.VMEM((C, HID), jnp.float32),
            pltpu.VMEM((C, HID), jnp.float32),
            pltpu.VMEM((C, HID), jnp.float32),
            pltpu.VMEM((DEMO_VOCAB * HID,), jnp.float32),
            pltpu.VMEM((MAX_POS * HID,), jnp.float32),
            pltpu.VMEM((2 * HID,), jnp.float32),
            pltpu.VMEM((HID,), jnp.float32),
            pltpu.VMEM((HID,), jnp.float32),
            pltpu.SemaphoreType.DMA,
            pltpu.SemaphoreType.DMA,
            pltpu.SemaphoreType.DMA,
            pltpu.SemaphoreType.DMA,
            pltpu.SemaphoreType.DMA,
            pltpu.SemaphoreType.DMA,
            pltpu.SemaphoreType.DMA,
            pltpu.SemaphoreType.DMA,
        ],
    )
    def k(idw2_h, ids5_h, wtab_h, dtab_h, ptab_h, stab_h, gamma_h, beta_h,
          out_h,
          idwb, idsb, wr0, wr1, wr2, wr3, dtab_v, ptab_v, stab_v, g_v, b_v,
          sg0, sg1, sg2, sg3, so0, so1, so2, so3):
        wid = lax.axis_index("s") * NC + lax.axis_index("c")
        row0 = wid * rows_w

        pltpu.sync_copy(dtab_h, dtab_v)
        pltpu.sync_copy(ptab_h, ptab_v)
        pltpu.sync_copy(stab_h, stab_v)
        pltpu.sync_copy(gamma_h, g_v)
        pltpu.sync_copy(beta_h, b_v)
        pltpu.sync_copy(idw2_h.at[wid], idwb)
        pltpu.sync_copy(ids5_h.at[wid], idsb)

        wrs = (wr0, wr1, wr2, wr3)
        sem_g = (sg0, sg1, sg2, sg3)
        sem_o = (so0, so1, so2, so3)
        AGE, BMI, CYC, SEG, POS = range(NS5)
        H4 = C // 4

        def issue_word(g, p):
            for q in range(4):
                pltpu.async_copy(
                    wtab_h.at[idwb.at[4 * g + q]],
                    wrs[p].at[pl.ds(q * H4, H4)], sem_g[p])

        def wait_word(g, p):
            for q in range(4):
                pltpu.make_async_copy(
                    wtab_h.at[idwb.at[4 * g + q]],
                    wrs[p].at[pl.ds(q * H4, H4)], sem_g[p]).wait()

        def issue_flush(g, p):
            pltpu.async_copy(wrs[p], out_h.at[row0 + g], sem_o[p])

        def wait_flush(p):
            pltpu.make_async_copy(wrs[p], out_h.at[row0], sem_o[p]).wait()

        def compute(g, p):
            wr = wrs[p]
            iota = lax.iota(jnp.int32, L16)
            cvec = [kk * L16 + iota for kk in range(KV)]
            lo8 = iota < 8
            gvec = [g_v[pl.ds(kk * L16, L16)] for kk in range(KV)]
            bvec = [b_v[pl.ds(kk * L16, L16)] for kk in range(KV)]
            s0vec = [stab_v[pl.ds(kk * L16, L16)] for kk in range(KV)]
            sdvec = [stab_v[pl.ds(HID + kk * L16, L16)] - s0vec[kk]
                     for kk in range(KV)]

            def grow(tab_v, idv, j):
                base = _bcast(idv, j)
                return [plsc.load_gather(tab_v, [base + cvec[kk]])
                        for kk in range(KV)]

            def embed(av, bv, cv, pv, sv, t, j):
                ar = grow(dtab_v, av, j)
                br = grow(dtab_v, bv, j)
                cr = grow(dtab_v, cv, j)
                pr = grow(ptab_v, pv, j)
                segf = _bcast(sv, j).astype(jnp.float32)
                acc = []
                for kk in range(KV):
                    v = ((wr[t, pl.ds(kk * L16, L16)] + ar[kk])
                         + (br[kk] + cr[kk])
                         + (pr[kk] + (s0vec[kk] + segf * sdvec[kk])))
                    acc.append(v)
                s1 = (acc[0] + acc[1]) + (acc[2] + acc[3])
                sq = ((acc[0] * acc[0] + acc[1] * acc[1])
                      + (acc[2] * acc[2] + acc[3] * acc[3]))
                return acc, s1, sq

            def fold2(xa, xb):
                m = jnp.where(lo8, xa + _xorp(xa, iota, 8),
                              xb + _xorp(xb, iota, 8))
                for kbit in (4, 2, 1):
                    m = m + _xorp(m, iota, kbit)
                return m

            @pl.loop(0, C // L16)
            def _grp(gg):
                s = gg * L16
                av = idsb[AGE * rows_w + g, pl.ds(s, L16)]
                bv = idsb[BMI * rows_w + g, pl.ds(s, L16)]
                cv = idsb[CYC * rows_w + g, pl.ds(s, L16)]
                sv = idsb[SEG * rows_w + g, pl.ds(s, L16)]
                pv = idsb[POS * rows_w + g, pl.ds(s, L16)]

                for j2 in range(L16 // 2):
                    ta, tb = s + 2 * j2, s + 2 * j2 + 1
                    acc_a, s1a, sqa = embed(av, bv, cv, pv, sv, ta, 2 * j2)
                    acc_b, s1b, sqb = embed(av, bv, cv, pv, sv, tb,
                                            2 * j2 + 1)
                    su = fold2(s1a, s1b)
                    qu = fold2(sqa, sqb)
                    mn = su * (1.0 / HID)
                    var = qu * (1.0 / HID) - mn * mn
                    rs = _rsqrt(var + 1e-12)
                    m_a, m_b = _bcast(mn, 0), _bcast(mn, 8)
                    r_a, r_b = _bcast(rs, 0), _bcast(rs, 8)
                    for kk in range(KV):
                        wr[ta, pl.ds(kk * L16, L16)] = (
                            (acc_a[kk] - m_a) * (r_a * gvec[kk]) + bvec[kk])
                        wr[tb, pl.ds(kk * L16, L16)] = (
                            (acc_b[kk] - m_b) * (r_b * gvec[kk]) + bvec[kk])

            del _grp

        def do_chunk(g, p, pr, steady):
            wait_word(g, p)
            compute(g, p)
            issue_flush(g, p)
            if steady:
                @pl.when(jnp.logical_and(g >= 1, g + NB - 1 < nchunk))
                def _():
                    wait_flush(pr)

                @pl.when(g + NB - 1 < nchunk)
                def _():
                    issue_word(g + NB - 1, pr)

        for p in range(NB - 1):
            issue_word(p, p)

        @pl.loop(0, (nchunk - 2) // NB)
        def _ring(i):
            g = i * NB
            for p in range(NB):
                do_chunk(g + p, p, (p - 1) % NB, True)

        do_chunk(nchunk - 2, 0, NB - 1, False)
        do_chunk(nchunk - 1, 1, NB - 1, False)

        for p in range(2, NB):
            wait_flush(p)
        wait_flush(0)
        wait_flush(1)

    return k(idw2, ids5, wtab, dtab_f, ptab_f, stab_f, gamma, beta)


def kernel(word_ids, age_ids, bmi_ids, cycle_len_ids, seg_ids, posi_ids,
           word_table, demo_table, posi_table, seg_table, ln_gamma, ln_beta):
    b, l = word_ids.shape
    n_tok = b * l
    rows_w = n_tok // (NW * C)
    idw2 = word_ids.reshape(NW, 4 * rows_w, C // 4).astype(jnp.int32)
    as_w = lambda x: x.reshape(NW, rows_w, C).astype(jnp.int32)
    ids5 = jnp.stack(
        [as_w(age_ids) * HID, as_w(bmi_ids) * HID,
         as_w(cycle_len_ids) * HID, as_w(seg_ids),
         as_w(posi_ids) * HID],
        axis=1).reshape(NW, NS5 * rows_w, C)
    out = _embed_ln(
        n_tok, idw2, ids5,
        word_table.astype(jnp.float32),
        demo_table.astype(jnp.float32).reshape(-1),
        posi_table.astype(jnp.float32).reshape(-1),
        seg_table.astype(jnp.float32).reshape(-1),
        ln_gamma.astype(jnp.float32), ln_beta.astype(jnp.float32),
    )
    return out.reshape(b, l, HID)
